# baseline (speedup 1.0000x reference)
import functools

import jax
import jax.numpy as jnp
from jax import lax
from jax.experimental import pallas as pl
from jax.experimental.pallas import tpu as pltpu
from jax.experimental.pallas import tpu_sc as plsc

NUM_STUDENTS = 1000000
NUM_QUESTIONS = 100000
NUM_DIM = 64
BATCH = 16384

NC = 2
NS = 16
L = 16
NW = NC * NS
B_PER_W = BATCH // NW
CHUNK = 128
N_CHUNKS = B_PER_W // CHUNK

ROWSUM_BW = 32768

TS_SC = 524288
TC_OFF = TS_SC // ROWSUM_BW
COLS_W = TS_SC // NW
SCW = 512
NCHK = COLS_W // SCW
KP = NUM_DIM // 8


def _rowsum_body(xt_ref, o_ref):
    ones = jnp.ones((1, NUM_DIM), jnp.float32)
    o_ref[...] = jnp.dot(ones, xt_ref[...],
                         preferred_element_type=jnp.float32)


def _rowsum_tc(xt, start_block):
    n = xt.shape[1]
    grid = (n + ROWSUM_BW - 1) // ROWSUM_BW - start_block
    return pl.pallas_call(
        _rowsum_body,
        grid=(grid,),
        in_specs=[pl.BlockSpec((NUM_DIM, ROWSUM_BW),
                               lambda i: (0, i + start_block))],
        out_specs=pl.BlockSpec((1, ROWSUM_BW), lambda i: (0, i + start_block)),
        out_shape=jax.ShapeDtypeStruct((1, n), jnp.float32),
        compiler_params=pltpu.CompilerParams(
            dimension_semantics=("arbitrary",)),
    )(xt)


def _sc_sweep_body(t3_hbm, ts_hbm, buf, out_v, sem):
    wid = lax.axis_index("s") * NC + lax.axis_index("c")
    base = pl.multiple_of(wid * COLS_W, SCW)

    def issue(ch, slot):
        col = pl.multiple_of(base + ch * SCW, SCW)
        for k in range(KP):
            pltpu.async_copy(t3_hbm.at[k, :, pl.ds(col, SCW)],
                             buf.at[slot, k], sem)

    def drain(slot):
        for k in range(KP):
            pltpu.make_async_copy(t3_hbm.at[0, :, pl.ds(0, SCW)],
                                  buf.at[slot, k], sem).wait()

    def compute(ch, slot):
        def g_body(g, _):
            vals = [buf[slot, k, r, pl.ds(g * L, L)]
                    for k in range(KP) for r in range(8)]
            while len(vals) > 1:
                vals = [a + b for a, b in zip(vals[::2], vals[1::2])]
            out_v[pl.ds(ch * SCW + g * L, L)] = vals[0]
            return 0
        lax.fori_loop(0, SCW // L, g_body, 0)

    issue(0, 0)

    def chunk_body(ch, _):
        slot = ch % 2

        @pl.when(ch + 1 < NCHK)
        def _():
            issue(ch + 1, 1 - slot)

        drain(slot)
        compute(ch, slot)
        return 0

    lax.fori_loop(0, NCHK, chunk_body, 0, unroll=2)

    pltpu.sync_copy(out_v, ts_hbm.at[pl.ds(wid * COLS_W, COLS_W)])


def _sc_sweep(theta3):
    kern = functools.partial(
        pl.kernel,
        mesh=plsc.VectorSubcoreMesh(core_axis_name="c", subcore_axis_name="s"),
        out_type=jax.ShapeDtypeStruct((TS_SC,), jnp.float32),
        compiler_params=pltpu.CompilerParams(
            needs_layout_passes=False, use_tc_tiling_on_sc=True),
        scratch_types=[
            pltpu.VMEM((2, KP, 8, SCW), jnp.float32),
            pltpu.VMEM((COLS_W,), jnp.float32),
            pltpu.SemaphoreType.DMA,
        ],
    )(_sc_sweep_body)
    return kern(theta3)


def _gather_body(sid_hbm, qid_hbm, tsc_hbm, ttc_hbm, bs_hbm, out_hbm,
                 sid_v, qid_v, scidx_v, tsc_v, ttc_v, bs_v, out_v, sem):
    wid = lax.axis_index("s") * NC + lax.axis_index("c")

    pltpu.sync_copy(sid_hbm.at[pl.ds(wid * N_CHUNKS, N_CHUNKS)], sid_v)
    pltpu.sync_copy(qid_hbm.at[pl.ds(wid * N_CHUNKS, N_CHUNKS)], qid_v)

    for j in range(N_CHUNKS):
        for c in range(CHUNK // L):
            s = sid_v[j, pl.ds(c * L, L)]
            scidx_v[j, pl.ds(c * L, L)] = jnp.minimum(s, TS_SC - 1)

    copies = []
    for j in range(N_CHUNKS):
        copies.append(pltpu.async_copy(tsc_hbm.at[scidx_v.at[j]],
                                       tsc_v.at[j], sem))
        copies.append(pltpu.async_copy(ttc_hbm.at[sid_v.at[j]],
                                       ttc_v.at[j], sem))
        copies.append(pltpu.async_copy(bs_hbm.at[qid_v.at[j]],
                                       bs_v.at[j], sem))
    for c in copies:
        c.wait()

    for j in range(N_CHUNKS):
        for c in range(CHUNK // L):
            s = sid_v[j, pl.ds(c * L, L)]
            ts = jnp.where(s < TS_SC,
                           tsc_v[j, pl.ds(c * L, L)],
                           ttc_v[j, pl.ds(c * L, L)])
            diff = ts - bs_v[j, pl.ds(c * L, L)]
            pred = 1.0 / (1.0 + jnp.exp(-diff))
            out_v[pl.ds((j * (CHUNK // L) + c) * L, L)] = pred

    pltpu.sync_copy(out_v, out_hbm.at[pl.ds(wid * B_PER_W, B_PER_W)])


def _gather_sigmoid(sid2d, qid2d, ts_sc, ts_tc, bs):
    kern = functools.partial(
        pl.kernel,
        mesh=plsc.VectorSubcoreMesh(core_axis_name="c", subcore_axis_name="s"),
        out_type=jax.ShapeDtypeStruct((BATCH,), jnp.float32),
        compiler_params=pltpu.CompilerParams(
            needs_layout_passes=False, use_tc_tiling_on_sc=False),
        scratch_types=[
            pltpu.VMEM((N_CHUNKS, CHUNK), jnp.int32),
            pltpu.VMEM((N_CHUNKS, CHUNK), jnp.int32),
            pltpu.VMEM((N_CHUNKS, CHUNK), jnp.int32),
            pltpu.VMEM((N_CHUNKS, CHUNK), jnp.float32),
            pltpu.VMEM((N_CHUNKS, CHUNK), jnp.float32),
            pltpu.VMEM((N_CHUNKS, CHUNK), jnp.float32),
            pltpu.VMEM((B_PER_W,), jnp.float32),
            pltpu.SemaphoreType.DMA,
        ],
    )(_gather_body)
    return kern(sid2d, qid2d, ts_sc, ts_tc, bs)


@jax.jit
def _irt(student_ids, question_ids, theta_weight, beta_weight):
    theta_t = theta_weight.T
    theta3 = theta_t.reshape(KP, 8, NUM_STUDENTS)
    ts_sc = _sc_sweep(theta3)
    ts_tc = _rowsum_tc(theta_t, TC_OFF).reshape(NUM_STUDENTS)
    bs = _rowsum_tc(beta_weight.T, 0).reshape(NUM_QUESTIONS)
    sid2d = student_ids.astype(jnp.int32).reshape(NW * N_CHUNKS, CHUNK)
    qid2d = question_ids.astype(jnp.int32).reshape(NW * N_CHUNKS, CHUNK)
    return _gather_sigmoid(sid2d, qid2d, ts_sc, ts_tc, bs)


def kernel(student_ids, question_ids, theta_weight, beta_weight):
    out = _irt(student_ids, question_ids, theta_weight, beta_weight)
    return out.reshape(BATCH, 1)


# speedup vs baseline: 1.6437x; 1.6437x over previous
import functools

import jax
import jax.numpy as jnp
from jax import lax
from jax.experimental import pallas as pl
from jax.experimental.pallas import tpu as pltpu
from jax.experimental.pallas import tpu_sc as plsc

NUM_STUDENTS = 1000000
NUM_QUESTIONS = 100000
NUM_DIM = 64
BATCH = 16384

NC = 2
NS = 16
L = 16
NW = NC * NS
B_PER_W = BATCH // NW
CHUNK = 128
N_CHUNKS = B_PER_W // CHUNK

ROWSUM_BW = 32768

TS_SC = 524288
TC_OFF = TS_SC // ROWSUM_BW
COLS_W = TS_SC // NW
SCW = 512
NCHK = COLS_W // SCW
KP = NUM_DIM // 8


def _rowsum_body(xt_ref, o_ref):
    ones = jnp.ones((1, NUM_DIM), jnp.float32)
    acc = jnp.dot(ones, xt_ref[...], preferred_element_type=jnp.float32)
    o_ref[...] = acc.reshape(ROWSUM_BW)


def _rowsum_tc(xt, start_block):
    n = xt.shape[1]
    grid = (n + ROWSUM_BW - 1) // ROWSUM_BW - start_block
    return pl.pallas_call(
        _rowsum_body,
        grid=(grid,),
        in_specs=[pl.BlockSpec((NUM_DIM, ROWSUM_BW),
                               lambda i: (0, i + start_block))],
        out_specs=pl.BlockSpec((ROWSUM_BW,), lambda i: (i + start_block,)),
        out_shape=jax.ShapeDtypeStruct((n,), jnp.float32),
        compiler_params=pltpu.CompilerParams(
            dimension_semantics=("arbitrary",)),
    )(xt)


def _sc_sweep_body(t3_hbm, ts_hbm, buf, out_v, sem):
    wid = lax.axis_index("s") * NC + lax.axis_index("c")
    base = pl.multiple_of(wid * COLS_W, SCW)

    def issue(ch, slot):
        col = pl.multiple_of(base + ch * SCW, SCW)
        pltpu.async_copy(t3_hbm.at[:, :, pl.ds(col, SCW)],
                         buf.at[slot], sem)

    def drain(slot):
        pltpu.make_async_copy(t3_hbm.at[:, :, pl.ds(0, SCW)],
                              buf.at[slot], sem).wait()

    def compute(ch, slot):
        def g_body(g, _):
            vals = [buf[slot, k, r, pl.ds(g * L, L)]
                    for k in range(KP) for r in range(8)]
            while len(vals) > 1:
                vals = [a + b for a, b in zip(vals[::2], vals[1::2])]
            out_v[pl.ds(ch * SCW + g * L, L)] = vals[0]
            return 0
        lax.fori_loop(0, SCW // L, g_body, 0)

    issue(0, 0)

    def chunk_body(ch, _):
        slot = ch % 2

        @pl.when(ch + 1 < NCHK)
        def _():
            issue(ch + 1, 1 - slot)

        drain(slot)
        compute(ch, slot)
        return 0

    lax.fori_loop(0, NCHK, chunk_body, 0, unroll=2)

    pltpu.sync_copy(out_v, ts_hbm.at[pl.ds(wid * COLS_W, COLS_W)])


def _sc_sweep(theta3):
    kern = functools.partial(
        pl.kernel,
        mesh=plsc.VectorSubcoreMesh(core_axis_name="c", subcore_axis_name="s"),
        out_type=jax.ShapeDtypeStruct((TS_SC,), jnp.float32),
        compiler_params=pltpu.CompilerParams(
            needs_layout_passes=False, use_tc_tiling_on_sc=True),
        scratch_types=[
            pltpu.VMEM((2, KP, 8, SCW), jnp.float32),
            pltpu.VMEM((COLS_W,), jnp.float32),
            pltpu.SemaphoreType.DMA,
        ],
    )(_sc_sweep_body)
    return kern(theta3)


def _gather_body(sid_hbm, qid_hbm, tsc_hbm, ttc_hbm, bs_hbm, out_hbm,
                 sid_v, qid_v, scidx_v, tsc_v, ttc_v, bs_v, out_v, sem):
    wid = lax.axis_index("s") * NC + lax.axis_index("c")

    pltpu.sync_copy(sid_hbm.at[pl.ds(wid * N_CHUNKS, N_CHUNKS)], sid_v)
    pltpu.sync_copy(qid_hbm.at[pl.ds(wid * N_CHUNKS, N_CHUNKS)], qid_v)

    for j in range(N_CHUNKS):
        for c in range(CHUNK // L):
            s = sid_v[j, pl.ds(c * L, L)]
            scidx_v[j, pl.ds(c * L, L)] = jnp.where(
                s < TS_SC, s, s - TS_SC)

    copies = []
    for j in range(N_CHUNKS):
        copies.append(pltpu.async_copy(tsc_hbm.at[scidx_v.at[j]],
                                       tsc_v.at[j], sem))
        copies.append(pltpu.async_copy(ttc_hbm.at[sid_v.at[j]],
                                       ttc_v.at[j], sem))
        copies.append(pltpu.async_copy(bs_hbm.at[qid_v.at[j]],
                                       bs_v.at[j], sem))
    for c in copies:
        c.wait()

    for j in range(N_CHUNKS):
        for c in range(CHUNK // L):
            s = sid_v[j, pl.ds(c * L, L)]
            ts = jnp.where(s < TS_SC,
                           tsc_v[j, pl.ds(c * L, L)],
                           ttc_v[j, pl.ds(c * L, L)])
            diff = ts - bs_v[j, pl.ds(c * L, L)]
            pred = 1.0 / (1.0 + jnp.exp(-diff))
            out_v[pl.ds((j * (CHUNK // L) + c) * L, L)] = pred

    pltpu.sync_copy(out_v, out_hbm.at[pl.ds(wid * B_PER_W, B_PER_W)])


def _gather_sigmoid(sid2d, qid2d, ts_sc, ts_tc, bs):
    kern = functools.partial(
        pl.kernel,
        mesh=plsc.VectorSubcoreMesh(core_axis_name="c", subcore_axis_name="s"),
        out_type=jax.ShapeDtypeStruct((BATCH,), jnp.float32),
        compiler_params=pltpu.CompilerParams(
            needs_layout_passes=False, use_tc_tiling_on_sc=False),
        scratch_types=[
            pltpu.VMEM((N_CHUNKS, CHUNK), jnp.int32),
            pltpu.VMEM((N_CHUNKS, CHUNK), jnp.int32),
            pltpu.VMEM((N_CHUNKS, CHUNK), jnp.int32),
            pltpu.VMEM((N_CHUNKS, CHUNK), jnp.float32),
            pltpu.VMEM((N_CHUNKS, CHUNK), jnp.float32),
            pltpu.VMEM((N_CHUNKS, CHUNK), jnp.float32),
            pltpu.VMEM((B_PER_W,), jnp.float32),
            pltpu.SemaphoreType.DMA,
        ],
    )(_gather_body)
    return kern(sid2d, qid2d, ts_sc, ts_tc, bs)


@jax.jit
def _irt(student_ids, question_ids, theta_weight, beta_weight):
    theta_t = theta_weight.T
    theta3 = theta_t.reshape(KP, 8, NUM_STUDENTS)
    ts_sc = _sc_sweep(theta3)
    ts_tc = _rowsum_tc(theta_t, TC_OFF)
    bs = _rowsum_tc(beta_weight.T, 0)
    sid2d = student_ids.astype(jnp.int32).reshape(NW * N_CHUNKS, CHUNK)
    qid2d = question_ids.astype(jnp.int32).reshape(NW * N_CHUNKS, CHUNK)
    return _gather_sigmoid(sid2d, qid2d, ts_sc, ts_tc, bs)


def kernel(student_ids, question_ids, theta_weight, beta_weight):
    out = _irt(student_ids, question_ids, theta_weight, beta_weight)
    return out.reshape(BATCH, 1)


# speedup vs baseline: 1.6558x; 1.0074x over previous
import functools

import jax
import jax.numpy as jnp
from jax import lax
from jax.experimental import pallas as pl
from jax.experimental.pallas import tpu as pltpu
from jax.experimental.pallas import tpu_sc as plsc

NUM_STUDENTS = 1000000
NUM_QUESTIONS = 100000
NUM_DIM = 64
BATCH = 16384

NC = 2
NS = 16
L = 16
NW = NC * NS
B_PER_W = BATCH // NW
CHUNK = 128
N_CHUNKS = B_PER_W // CHUNK

ROWSUM_BW = 16384

TS_SC = 524288
TC_OFF = TS_SC // ROWSUM_BW
COLS_W = TS_SC // NW
SCW = 512
NCHK = COLS_W // SCW
KP = NUM_DIM // 8


def _rowsum_body(xt_ref, o_ref):
    ones = jnp.ones((1, NUM_DIM), jnp.float32)
    acc = jnp.dot(ones, xt_ref[...], preferred_element_type=jnp.float32)
    o_ref[...] = acc.reshape(ROWSUM_BW)


def _rowsum_tc(xt, start_block):
    n = xt.shape[1]
    grid = (n + ROWSUM_BW - 1) // ROWSUM_BW - start_block
    return pl.pallas_call(
        _rowsum_body,
        grid=(grid,),
        in_specs=[pl.BlockSpec((NUM_DIM, ROWSUM_BW),
                               lambda i: (0, i + start_block))],
        out_specs=pl.BlockSpec((ROWSUM_BW,), lambda i: (i + start_block,)),
        out_shape=jax.ShapeDtypeStruct((n,), jnp.float32),
        compiler_params=pltpu.CompilerParams(
            dimension_semantics=("arbitrary",)),
    )(xt)


def _sc_sweep_body(t3_hbm, ts_hbm, buf, out_v, sem):
    wid = lax.axis_index("s") * NC + lax.axis_index("c")
    base = pl.multiple_of(wid * COLS_W, SCW)

    def issue(ch, slot):
        col = pl.multiple_of(base + ch * SCW, SCW)
        pltpu.async_copy(t3_hbm.at[:, :, pl.ds(col, SCW)],
                         buf.at[slot], sem)

    def drain(slot):
        pltpu.make_async_copy(t3_hbm.at[:, :, pl.ds(0, SCW)],
                              buf.at[slot], sem).wait()

    def compute(ch, slot):
        def g_body(g, _):
            vals = [buf[slot, k, r, pl.ds(g * L, L)]
                    for k in range(KP) for r in range(8)]
            while len(vals) > 1:
                vals = [a + b for a, b in zip(vals[::2], vals[1::2])]
            out_v[pl.ds(ch * SCW + g * L, L)] = vals[0]
            return 0
        lax.fori_loop(0, SCW // L, g_body, 0)

    issue(0, 0)

    def chunk_body(ch, _):
        slot = ch % 2

        @pl.when(ch + 1 < NCHK)
        def _():
            issue(ch + 1, 1 - slot)

        drain(slot)
        compute(ch, slot)
        return 0

    lax.fori_loop(0, NCHK, chunk_body, 0, unroll=2)

    pltpu.sync_copy(out_v, ts_hbm.at[pl.ds(wid * COLS_W, COLS_W)])


def _sc_sweep(theta3):
    kern = functools.partial(
        pl.kernel,
        mesh=plsc.VectorSubcoreMesh(core_axis_name="c", subcore_axis_name="s"),
        out_type=jax.ShapeDtypeStruct((TS_SC,), jnp.float32),
        compiler_params=pltpu.CompilerParams(
            needs_layout_passes=False, use_tc_tiling_on_sc=True),
        scratch_types=[
            pltpu.VMEM((2, KP, 8, SCW), jnp.float32),
            pltpu.VMEM((COLS_W,), jnp.float32),
            pltpu.SemaphoreType.DMA,
        ],
    )(_sc_sweep_body)
    return kern(theta3)


def _gather_body(sid_hbm, qid_hbm, tsc_hbm, ttc_hbm, bs_hbm, out_hbm,
                 sid_v, qid_v, scidx_v, tsc_v, ttc_v, bs_v, out_v, sem):
    wid = lax.axis_index("s") * NC + lax.axis_index("c")

    pltpu.sync_copy(sid_hbm.at[pl.ds(wid * N_CHUNKS, N_CHUNKS)], sid_v)
    pltpu.sync_copy(qid_hbm.at[pl.ds(wid * N_CHUNKS, N_CHUNKS)], qid_v)

    for j in range(N_CHUNKS):
        for c in range(CHUNK // L):
            s = sid_v[j, pl.ds(c * L, L)]
            scidx_v[j, pl.ds(c * L, L)] = jnp.where(
                s < TS_SC, s, s - TS_SC)

    copies = []
    for j in range(N_CHUNKS):
        copies.append(pltpu.async_copy(tsc_hbm.at[scidx_v.at[j]],
                                       tsc_v.at[j], sem))
        copies.append(pltpu.async_copy(ttc_hbm.at[sid_v.at[j]],
                                       ttc_v.at[j], sem))
        copies.append(pltpu.async_copy(bs_hbm.at[qid_v.at[j]],
                                       bs_v.at[j], sem))
    for c in copies:
        c.wait()

    for j in range(N_CHUNKS):
        for c in range(CHUNK // L):
            s = sid_v[j, pl.ds(c * L, L)]
            ts = jnp.where(s < TS_SC,
                           tsc_v[j, pl.ds(c * L, L)],
                           ttc_v[j, pl.ds(c * L, L)])
            diff = ts - bs_v[j, pl.ds(c * L, L)]
            pred = 1.0 / (1.0 + jnp.exp(-diff))
            out_v[pl.ds((j * (CHUNK // L) + c) * L, L)] = pred

    pltpu.sync_copy(out_v, out_hbm.at[pl.ds(wid * B_PER_W, B_PER_W)])


def _gather_sigmoid(sid2d, qid2d, ts_sc, ts_tc, bs):
    kern = functools.partial(
        pl.kernel,
        mesh=plsc.VectorSubcoreMesh(core_axis_name="c", subcore_axis_name="s"),
        out_type=jax.ShapeDtypeStruct((BATCH,), jnp.float32),
        compiler_params=pltpu.CompilerParams(
            needs_layout_passes=False, use_tc_tiling_on_sc=False),
        scratch_types=[
            pltpu.VMEM((N_CHUNKS, CHUNK), jnp.int32),
            pltpu.VMEM((N_CHUNKS, CHUNK), jnp.int32),
            pltpu.VMEM((N_CHUNKS, CHUNK), jnp.int32),
            pltpu.VMEM((N_CHUNKS, CHUNK), jnp.float32),
            pltpu.VMEM((N_CHUNKS, CHUNK), jnp.float32),
            pltpu.VMEM((N_CHUNKS, CHUNK), jnp.float32),
            pltpu.VMEM((B_PER_W,), jnp.float32),
            pltpu.SemaphoreType.DMA,
        ],
    )(_gather_body)
    return kern(sid2d, qid2d, ts_sc, ts_tc, bs)


@jax.jit
def _irt(student_ids, question_ids, theta_weight, beta_weight):
    theta_t = theta_weight.T
    theta3 = theta_t.reshape(KP, 8, NUM_STUDENTS)
    ts_sc = _sc_sweep(theta3)
    ts_tc = _rowsum_tc(theta_t, TC_OFF)
    bs = _rowsum_tc(beta_weight.T, 0)
    sid2d = student_ids.astype(jnp.int32).reshape(NW * N_CHUNKS, CHUNK)
    qid2d = question_ids.astype(jnp.int32).reshape(NW * N_CHUNKS, CHUNK)
    return _gather_sigmoid(sid2d, qid2d, ts_sc, ts_tc, bs)


def kernel(student_ids, question_ids, theta_weight, beta_weight):
    out = _irt(student_ids, question_ids, theta_weight, beta_weight)
    return out.reshape(BATCH, 1)
